# Initial kernel scaffold; baseline (speedup 1.0000x reference)
import jax
import jax.numpy as jnp
from jax.experimental import pallas as pl


def kernel(x, edge_index, batch, device, W0, b0, W1, b1, W2, b2, Wfc, bfc, Wcls, bcls):
    raise NotImplementedError("write your pallas kernel here")


# speedup vs baseline: 11.5656x; 11.5656x over previous
import functools

import jax
import jax.numpy as jnp
from jax import lax
from jax.experimental import pallas as pl
from jax.experimental.pallas import tpu as pltpu
from jax.experimental.pallas import tpu_sc as plsc

N = 10000
E = 320000
F_IN = 128
HID = 64
NG = 128
NCLS = 10

NPAD = 10240
ROWS_PER_TILE = NPAD // 16
BLK = 128
NBLK = 79
EPW = NBLK * BLK
NWORK = 32
E_PAD = NWORK * EPW
CW = 16

_mesh = plsc.VectorSubcoreMesh(core_axis_name="c", subcore_axis_name="s")
_sc_params = pltpu.CompilerParams(use_tc_tiling_on_sc=False)


@functools.partial(
    pl.kernel,
    out_type=jax.ShapeDtypeStruct((2, NPAD, CW), jnp.float32),
    mesh=_mesh,
    compiler_params=_sc_params,
    scratch_types=[
        pltpu.VMEM((BLK,), jnp.int32),
        pltpu.VMEM((BLK, CW), jnp.float32),
        pltpu.VMEM((BLK, CW), jnp.float32),
        pltpu.VMEM_SHARED((NPAD, CW), jnp.float32),
    ],
)
def _count_kernel(d_hbm, out_hbm, d_v, ones_v, z_v, acc_sh):
    cid = lax.axis_index("c")
    sid = lax.axis_index("s")
    wid = cid * 16 + sid

    @pl.loop(0, BLK)
    def _(r):
        ones_v[r, :] = jnp.ones((CW,), jnp.float32)
        z_v[r, :] = jnp.zeros((CW,), jnp.float32)

    @pl.loop(0, ROWS_PER_TILE // BLK)
    def _(j):
        pltpu.sync_copy(z_v, acc_sh.at[pl.ds(sid * ROWS_PER_TILE + j * BLK, BLK)])

    plsc.subcore_barrier()
    base = wid * EPW

    @pl.loop(0, NBLK)
    def _(b):
        off = pl.multiple_of(base + b * BLK, 8)
        pltpu.sync_copy(d_hbm.at[pl.ds(off, BLK)], d_v)
        pltpu.sync_copy(ones_v, acc_sh.at[d_v], add=True)

    plsc.subcore_barrier()

    @pl.loop(0, ROWS_PER_TILE // BLK)
    def _(j):
        r0 = sid * ROWS_PER_TILE + j * BLK
        pltpu.sync_copy(acc_sh.at[pl.ds(r0, BLK)], out_hbm.at[cid, pl.ds(r0, BLK)])


@functools.partial(
    pl.kernel,
    out_type=jax.ShapeDtypeStruct((2, NPAD, HID), jnp.float32),
    mesh=_mesh,
    compiler_params=_sc_params,
    scratch_types=[
        pltpu.VMEM((BLK,), jnp.int32),
        pltpu.VMEM((BLK,), jnp.int32),
        pltpu.VMEM((BLK, HID), jnp.float32),
        pltpu.VMEM((BLK, HID), jnp.float32),
        pltpu.VMEM_SHARED((NPAD, HID), jnp.float32),
        pltpu.SemaphoreType.DMA,
    ],
)
def _edge_kernel(hs_hbm, s_hbm, d_hbm, out_hbm, s_v, d_v, rows_v, z_v, acc_sh, sem):
    cid = lax.axis_index("c")
    sid = lax.axis_index("s")
    wid = cid * 16 + sid

    @pl.loop(0, BLK)
    def _(r):
        for c4 in range(HID // 16):
            z_v[r, pl.ds(c4 * 16, 16)] = jnp.zeros((16,), jnp.float32)

    @pl.loop(0, ROWS_PER_TILE // BLK)
    def _(j):
        pltpu.sync_copy(z_v, acc_sh.at[pl.ds(sid * ROWS_PER_TILE + j * BLK, BLK)])

    plsc.subcore_barrier()
    base = wid * EPW

    @pl.loop(0, NBLK)
    def _(b):
        off = pl.multiple_of(base + b * BLK, 8)
        pltpu.sync_copy(s_hbm.at[pl.ds(off, BLK)], s_v)
        pltpu.sync_copy(d_hbm.at[pl.ds(off, BLK)], d_v)
        pltpu.async_copy(hs_hbm.at[s_v], rows_v, sem).wait()
        pltpu.sync_copy(rows_v, acc_sh.at[d_v], add=True)

    plsc.subcore_barrier()

    @pl.loop(0, ROWS_PER_TILE // BLK)
    def _(j):
        r0 = sid * ROWS_PER_TILE + j * BLK
        pltpu.sync_copy(acc_sh.at[pl.ds(r0, BLK)], out_hbm.at[cid, pl.ds(r0, BLK)])


def _bn(t):
    m = jnp.mean(t, axis=0, keepdims=True)
    v = jnp.mean((t - m) ** 2, axis=0, keepdims=True)
    return (t - m) * lax.rsqrt(v + 1e-5) + 1e-4


def _tc0_body(x_ref, w_ref, c0_ref, c1_ref, hs_ref, dis_ref):
    z = _bn(x_ref[...])
    h = jnp.dot(z, w_ref[...], preferred_element_type=jnp.float32)
    deg = c0_ref[...] + c1_ref[...] + 1.0
    dis = lax.rsqrt(deg)
    dis_ref[...] = dis
    hs_ref[...] = h * dis


def _tc0(x, W0, cnt0, cnt1):
    return pl.pallas_call(
        _tc0_body,
        out_shape=(
            jax.ShapeDtypeStruct((N, HID), jnp.float32),
            jax.ShapeDtypeStruct((N, 1), jnp.float32),
        ),
    )(x, W0, cnt0, cnt1)


def _tcmid_body(p_ref, hs_ref, dis_ref, b_ref, w_ref, out_ref):
    acc = p_ref[0, :N, :] + p_ref[1, :N, :] + hs_ref[...]
    a = jnp.maximum(dis_ref[...] * acc + b_ref[...], 0.0)
    z = _bn(a)
    h = jnp.dot(z, w_ref[...], preferred_element_type=jnp.float32)
    out_ref[...] = h * dis_ref[...]


def _tcmid(p, hs, dis, b, W):
    return pl.pallas_call(
        _tcmid_body,
        out_shape=jax.ShapeDtypeStruct((N, HID), jnp.float32),
    )(p, hs, dis, b, W)


def _tcend_body(p_ref, hs_ref, dis_ref, b_ref, batch_ref, wfc_ref, bfc_ref,
                wcls_ref, bcls_ref, out_ref):
    acc = p_ref[0, :N, :] + p_ref[1, :N, :] + hs_ref[...]
    a = jnp.maximum(dis_ref[...] * acc + b_ref[...], 0.0)
    gids = lax.broadcasted_iota(jnp.int32, (NG, 1), 0)
    onehot = (gids == batch_ref[...]).astype(jnp.float32)
    g = jnp.dot(onehot, a, preferred_element_type=jnp.float32)
    g = _bn(g)
    g = jnp.maximum(
        jnp.dot(g, wfc_ref[...], preferred_element_type=jnp.float32) + bfc_ref[...],
        0.0,
    )
    g = _bn(g)
    g = jnp.dot(g, wcls_ref[...], preferred_element_type=jnp.float32) + bcls_ref[...]
    mx = jnp.max(g, axis=-1, keepdims=True)
    out_ref[...] = g - mx - jnp.log(jnp.sum(jnp.exp(g - mx), axis=-1, keepdims=True))


def _tcend(p, hs, dis, b, batch_row, Wfc, bfc, Wcls, bcls):
    return pl.pallas_call(
        _tcend_body,
        out_shape=jax.ShapeDtypeStruct((NG, NCLS), jnp.float32),
    )(p, hs, dis, b, batch_row, Wfc, bfc, Wcls, bcls)


def kernel(x, edge_index, batch, device, W0, b0, W1, b1, W2, b2, Wfc, bfc, Wcls, bcls):
    src = edge_index[0]
    dst = edge_index[1]
    pad = E_PAD - E
    s_pad = jnp.concatenate([src, jnp.zeros((pad,), jnp.int32)])
    d_pad = jnp.concatenate([dst, jnp.full((pad,), N, jnp.int32)])

    cnt = _count_kernel(d_pad)
    cnt0 = cnt[0, :N, 0:1]
    cnt1 = cnt[1, :N, 0:1]

    hs0, dis = _tc0(x, W0, cnt0, cnt1)
    p0 = _edge_kernel(hs0, s_pad, d_pad)
    hs1 = _tcmid(p0, hs0, dis, b0.reshape(1, -1), W1)
    p1 = _edge_kernel(hs1, s_pad, d_pad)
    hs2 = _tcmid(p1, hs1, dis, b1.reshape(1, -1), W2)
    p2 = _edge_kernel(hs2, s_pad, d_pad)
    return _tcend(p2, hs2, dis, b2.reshape(1, -1), batch.reshape(1, -1),
                  Wfc, bfc.reshape(1, -1), Wcls, bcls.reshape(1, -1))
